# baseline (speedup 1.0000x reference)
import math

import jax
import jax.numpy as jnp
from jax.experimental import pallas as pl
from jax.experimental.pallas import tpu as pltpu

DECAY = 0.99
TOP_K = 8
CHUNK = 16


def _proj_kernel(mem_ref, wk_ref, imp_ref, age_ref, kp_ref, bias_ref):
    kp_ref[...] = jnp.dot(mem_ref[...], wk_ref[...],
                          preferred_element_type=jnp.float32)
    eff = imp_ref[...] * jnp.exp(age_ref[...] * math.log(DECAY))
    bias_ref[...] = jnp.maximum(jnp.log(eff), -10.0)


def _attn_kernel(q_ref, wq_ref, kp_ref, bias_ref, mem_ref, attn_ref, ret_ref,
                 s_ref):
    tl = q_ref.shape[1]
    d = q_ref.shape[-1]
    qp = jnp.dot(q_ref[0], wq_ref[...], preferred_element_type=jnp.float32)
    s = jax.lax.dot_general(qp, kp_ref[...], (((1,), (1,)), ((), ())),
                            preferred_element_type=jnp.float32)
    s_ref[...] = s * (1.0 / math.sqrt(d)) + bias_ref[...]

    n_slots = s.shape[-1]
    neg_inf = jnp.float32(-jnp.inf)

    def chunk_body(c, n_sel):
        rows = pl.ds(c * CHUNK, CHUNK)
        sc = s_ref[rows, :]
        work = sc
        m0 = None
        for i in range(TOP_K):
            m = jnp.max(work, axis=1, keepdims=True)
            if i == 0:
                m0 = m
            work = jnp.where(work == m, neg_inf, work)
        selected = work == neg_inf
        e = jnp.where(selected, jnp.exp(sc - m0), 0.0)
        attn_ref[0, rows, :] = e / jnp.sum(e, axis=1, keepdims=True)
        return n_sel + jnp.sum(selected.astype(jnp.float32))

    n_sel = jax.lax.fori_loop(0, tl // CHUNK, chunk_body, jnp.float32(0.0))

    @pl.when(n_sel != float(TOP_K * tl))
    def _exact_repair():
        iota = jax.lax.broadcasted_iota(jnp.int32, (tl, n_slots), 1)
        sc = s_ref[...]
        work = sc
        for _ in range(TOP_K):
            m = jnp.max(work, axis=1, keepdims=True)
            first = jnp.min(jnp.where(work == m, iota, n_slots), axis=1,
                            keepdims=True)
            work = jnp.where(iota == first, neg_inf, work)
        m0 = jnp.max(sc, axis=1, keepdims=True)
        e = jnp.where(work == neg_inf, jnp.exp(sc - m0), 0.0)
        attn_ref[0] = e / jnp.sum(e, axis=1, keepdims=True)

    ret_ref[0] = jnp.dot(attn_ref[0], mem_ref[...],
                         preferred_element_type=jnp.float32)


def kernel(query, memory, importance, age, W_q, W_k, top_k):
    B, L, d = query.shape
    S = memory.shape[1]
    mem2d = memory.reshape(S, d)

    kp, bias = pl.pallas_call(
        _proj_kernel,
        out_shape=[
            jax.ShapeDtypeStruct((S, d), jnp.float32),
            jax.ShapeDtypeStruct((1, S), jnp.float32),
        ],
    )(mem2d, W_k, importance, age)

    tl = min(512, L)
    grid = (B, L // tl)
    attn, ret = pl.pallas_call(
        _attn_kernel,
        grid=grid,
        in_specs=[
            pl.BlockSpec((1, tl, d), lambda b, l: (b, l, 0)),
            pl.BlockSpec((d, d), lambda b, l: (0, 0)),
            pl.BlockSpec((S, d), lambda b, l: (0, 0)),
            pl.BlockSpec((1, S), lambda b, l: (0, 0)),
            pl.BlockSpec((S, d), lambda b, l: (0, 0)),
        ],
        out_specs=[
            pl.BlockSpec((1, tl, S), lambda b, l: (b, l, 0)),
            pl.BlockSpec((1, tl, d), lambda b, l: (b, l, 0)),
        ],
        out_shape=[
            jax.ShapeDtypeStruct((B, L, S), jnp.float32),
            jax.ShapeDtypeStruct((B, L, d), jnp.float32),
        ],
        scratch_shapes=[
            pltpu.VMEM((tl, S), jnp.float32),
        ],
        compiler_params=pltpu.CompilerParams(
            dimension_semantics=("parallel", "parallel")),
    )(query, W_q, kp, bias, mem2d)
    return ret, attn


# speedup vs baseline: 6.5012x; 6.5012x over previous
import math

import jax
import jax.numpy as jnp
from jax.experimental import pallas as pl
from jax.experimental.pallas import tpu as pltpu

DECAY = 0.99
TOP_K = 8


def _proj_kernel(mem_ref, wk_ref, imp_ref, age_ref, kp_ref, bias_ref):
    kp_ref[...] = jnp.dot(mem_ref[...], wk_ref[...],
                          preferred_element_type=jnp.float32)
    eff = imp_ref[...] * jnp.exp(age_ref[...] * math.log(DECAY))
    bias_ref[...] = jnp.maximum(jnp.log(eff), -10.0)


def _attn_kernel(q_ref, wq_ref, kp_ref, bias_ref, mem_ref, attn_ref, ret_ref,
                 s_ref):
    tl = q_ref.shape[1]
    d = q_ref.shape[-1]
    scale = 1.0 / math.sqrt(d)
    qp = jnp.dot(q_ref[0], wq_ref[...], preferred_element_type=jnp.float32)
    s = jax.lax.dot_general(qp, kp_ref[...], (((1,), (1,)), ((), ())),
                            preferred_element_type=jnp.float32)
    s = s + bias_ref[...] * math.sqrt(d)
    s_ref[...] = s

    n_slots = s.shape[-1]
    neg_inf = jnp.float32(-jnp.inf)

    work = s
    ms = []
    for _ in range(TOP_K):
        m = jnp.max(work, axis=1, keepdims=True)
        ms.append(m)
        work = jnp.where(work == m, neg_inf, work)
    m0 = ms[0]
    denom = jnp.zeros_like(m0)
    for m in ms:
        denom = denom + jnp.exp((m - m0) * scale)
    rdenom = 1.0 / denom
    sel = work == neg_inf
    attn_ref[0] = jnp.where(sel, jnp.exp((s - m0) * scale) * rdenom, 0.0)
    n_sel = jnp.sum(sel.astype(jnp.float32))

    @pl.when(n_sel != float(TOP_K * tl))
    def _exact_repair():
        iota = jax.lax.broadcasted_iota(jnp.int32, (tl, n_slots), 1)
        sc = s_ref[...]
        work2 = sc
        for _ in range(TOP_K):
            m = jnp.max(work2, axis=1, keepdims=True)
            first = jnp.min(jnp.where(work2 == m, iota, n_slots), axis=1,
                            keepdims=True)
            work2 = jnp.where(iota == first, neg_inf, work2)
        mr = jnp.max(sc, axis=1, keepdims=True)
        e = jnp.where(work2 == neg_inf, jnp.exp((sc - mr) * scale), 0.0)
        attn_ref[0] = e / jnp.sum(e, axis=1, keepdims=True)

    ret_ref[0] = jnp.dot(attn_ref[0], mem_ref[...],
                         preferred_element_type=jnp.float32)


def kernel(query, memory, importance, age, W_q, W_k, top_k):
    B, L, d = query.shape
    S = memory.shape[1]
    mem2d = memory.reshape(S, d)

    kp, bias = pl.pallas_call(
        _proj_kernel,
        out_shape=[
            jax.ShapeDtypeStruct((S, d), jnp.float32),
            jax.ShapeDtypeStruct((1, S), jnp.float32),
        ],
    )(mem2d, W_k, importance, age)

    tl = min(512, L)
    grid = (B, L // tl)
    attn, ret = pl.pallas_call(
        _attn_kernel,
        grid=grid,
        in_specs=[
            pl.BlockSpec((1, tl, d), lambda b, l: (b, l, 0)),
            pl.BlockSpec((d, d), lambda b, l: (0, 0)),
            pl.BlockSpec((S, d), lambda b, l: (0, 0)),
            pl.BlockSpec((1, S), lambda b, l: (0, 0)),
            pl.BlockSpec((S, d), lambda b, l: (0, 0)),
        ],
        out_specs=[
            pl.BlockSpec((1, tl, S), lambda b, l: (b, l, 0)),
            pl.BlockSpec((1, tl, d), lambda b, l: (b, l, 0)),
        ],
        out_shape=[
            jax.ShapeDtypeStruct((B, L, S), jnp.float32),
            jax.ShapeDtypeStruct((B, L, d), jnp.float32),
        ],
        scratch_shapes=[
            pltpu.VMEM((tl, S), jnp.float32),
        ],
        compiler_params=pltpu.CompilerParams(
            dimension_semantics=("parallel", "parallel")),
    )(query, W_q, kp, bias, mem2d)
    return ret, attn


# speedup vs baseline: 7.2950x; 1.1221x over previous
import math

import jax
import jax.numpy as jnp
from jax.experimental import pallas as pl
from jax.experimental.pallas import tpu as pltpu

DECAY = 0.99
TOP_K = 8


def _proj_kernel(mem_ref, wk_ref, imp_ref, age_ref, kp_ref, bias_ref):
    kp_ref[...] = jnp.dot(mem_ref[...], wk_ref[...],
                          preferred_element_type=jnp.float32)
    eff = imp_ref[...] * jnp.exp(age_ref[...] * math.log(DECAY))
    bias_ref[...] = jnp.maximum(jnp.log(eff), -10.0)


def _attn_kernel(q_ref, wq_ref, kp_ref, bias_ref, mem_ref, attn_ref, ret_ref,
                 s_ref, w_ref):
    tl = q_ref.shape[1]
    d = q_ref.shape[-1]
    qp = jnp.dot(q_ref[0], wq_ref[...], preferred_element_type=jnp.float32)
    s = jax.lax.dot_general(qp, kp_ref[...], (((1,), (1,)), ((), ())),
                            preferred_element_type=jnp.float32)
    s = s * (1.0 / math.sqrt(d)) + bias_ref[...]
    s_ref[...] = s

    n_slots = s.shape[-1]
    neg_inf = jnp.float32(-jnp.inf)

    work = s
    m0 = None
    for i in range(TOP_K):
        m = jnp.max(work, axis=1, keepdims=True)
        if i == 0:
            m0 = m
        work = jnp.where(work == m, neg_inf, work)
    w_ref[...] = work
    n_sel = jnp.sum((work == neg_inf).astype(jnp.float32))

    @pl.when(n_sel != float(TOP_K * tl))
    def _exact_repair():
        iota = jax.lax.broadcasted_iota(jnp.int32, (tl, n_slots), 1)
        work2 = s_ref[...]
        for _ in range(TOP_K):
            m = jnp.max(work2, axis=1, keepdims=True)
            first = jnp.min(jnp.where(work2 == m, iota, n_slots), axis=1,
                            keepdims=True)
            work2 = jnp.where(iota == first, neg_inf, work2)
        w_ref[...] = work2

    sel = w_ref[...] == neg_inf
    e = jnp.where(sel, jnp.exp(s_ref[...] - m0), 0.0)
    attn = e / jnp.sum(e, axis=1, keepdims=True)
    attn_ref[0] = attn
    ret_ref[0] = jnp.dot(attn, mem_ref[...],
                         preferred_element_type=jnp.float32)


def kernel(query, memory, importance, age, W_q, W_k, top_k):
    B, L, d = query.shape
    S = memory.shape[1]
    mem2d = memory.reshape(S, d)

    kp, bias = pl.pallas_call(
        _proj_kernel,
        out_shape=[
            jax.ShapeDtypeStruct((S, d), jnp.float32),
            jax.ShapeDtypeStruct((1, S), jnp.float32),
        ],
    )(mem2d, W_k, importance, age)

    tl = min(1024, L)
    grid = (B, L // tl)
    attn, ret = pl.pallas_call(
        _attn_kernel,
        grid=grid,
        in_specs=[
            pl.BlockSpec((1, tl, d), lambda b, l: (b, l, 0)),
            pl.BlockSpec((d, d), lambda b, l: (0, 0)),
            pl.BlockSpec((S, d), lambda b, l: (0, 0)),
            pl.BlockSpec((1, S), lambda b, l: (0, 0)),
            pl.BlockSpec((S, d), lambda b, l: (0, 0)),
        ],
        out_specs=[
            pl.BlockSpec((1, tl, S), lambda b, l: (b, l, 0)),
            pl.BlockSpec((1, tl, d), lambda b, l: (b, l, 0)),
        ],
        out_shape=[
            jax.ShapeDtypeStruct((B, L, S), jnp.float32),
            jax.ShapeDtypeStruct((B, L, d), jnp.float32),
        ],
        scratch_shapes=[
            pltpu.VMEM((tl, S), jnp.float32),
            pltpu.VMEM((tl, S), jnp.float32),
        ],
        compiler_params=pltpu.CompilerParams(
            dimension_semantics=("parallel", "parallel")),
    )(query, W_q, kp, bias, mem2d)
    return ret, attn


# speedup vs baseline: 7.6664x; 1.0509x over previous
import math

import jax
import jax.numpy as jnp
from jax.experimental import pallas as pl
from jax.experimental.pallas import tpu as pltpu

DECAY = 0.99
TOP_K = 8


def _proj_kernel(mem_ref, wk_ref, imp_ref, age_ref, kp_ref, bias_ref):
    kp_ref[...] = jnp.dot(mem_ref[...], wk_ref[...],
                          preferred_element_type=jnp.float32)
    eff = imp_ref[...] * jnp.exp(age_ref[...] * math.log(DECAY))
    bias_ref[...] = jnp.maximum(jnp.log(eff), -10.0)


def _attn_kernel(q_ref, wq_ref, kp_ref, bias_ref, mem_ref, attn_ref, ret_ref,
                 s_ref, w_ref):
    tl = q_ref.shape[1]
    d = q_ref.shape[-1]
    qp = jnp.dot(q_ref[0], wq_ref[...], preferred_element_type=jnp.float32)
    s = jax.lax.dot_general(qp, kp_ref[...], (((1,), (1,)), ((), ())),
                            preferred_element_type=jnp.float32)
    s = s * (1.0 / math.sqrt(d)) + bias_ref[...]
    s_ref[...] = s

    n_slots = s.shape[-1]
    neg_inf = jnp.float32(-jnp.inf)

    work = s
    m0 = None
    for i in range(TOP_K):
        m = jnp.max(work, axis=1, keepdims=True)
        if i == 0:
            m0 = m
        work = jnp.where(work == m, neg_inf, work)
    w_ref[...] = work
    n_sel = jnp.sum((work == neg_inf).astype(jnp.float32))

    @pl.when(n_sel != float(TOP_K * tl))
    def _exact_repair():
        iota = jax.lax.broadcasted_iota(jnp.int32, (tl, n_slots), 1)
        work2 = s_ref[...]
        for _ in range(TOP_K):
            m = jnp.max(work2, axis=1, keepdims=True)
            first = jnp.min(jnp.where(work2 == m, iota, n_slots), axis=1,
                            keepdims=True)
            work2 = jnp.where(iota == first, neg_inf, work2)
        w_ref[...] = work2

    sel = w_ref[...] == neg_inf
    e = jnp.where(sel, jnp.exp(s_ref[...] - m0), 0.0)
    attn = e / jnp.sum(e, axis=1, keepdims=True)
    attn_ref[0] = attn
    ret_ref[0] = jnp.dot(attn, mem_ref[...],
                         preferred_element_type=jnp.float32)


def kernel(query, memory, importance, age, W_q, W_k, top_k):
    B, L, d = query.shape
    S = memory.shape[1]
    mem2d = memory.reshape(S, d)

    kp, bias = pl.pallas_call(
        _proj_kernel,
        out_shape=[
            jax.ShapeDtypeStruct((S, d), jnp.float32),
            jax.ShapeDtypeStruct((1, S), jnp.float32),
        ],
    )(mem2d, W_k, importance, age)

    tl = min(2048, L)
    grid = (B, L // tl)
    attn, ret = pl.pallas_call(
        _attn_kernel,
        grid=grid,
        in_specs=[
            pl.BlockSpec((1, tl, d), lambda b, l: (b, l, 0)),
            pl.BlockSpec((d, d), lambda b, l: (0, 0)),
            pl.BlockSpec((S, d), lambda b, l: (0, 0)),
            pl.BlockSpec((1, S), lambda b, l: (0, 0)),
            pl.BlockSpec((S, d), lambda b, l: (0, 0)),
        ],
        out_specs=[
            pl.BlockSpec((1, tl, S), lambda b, l: (b, l, 0)),
            pl.BlockSpec((1, tl, d), lambda b, l: (b, l, 0)),
        ],
        out_shape=[
            jax.ShapeDtypeStruct((B, L, S), jnp.float32),
            jax.ShapeDtypeStruct((B, L, d), jnp.float32),
        ],
        scratch_shapes=[
            pltpu.VMEM((tl, S), jnp.float32),
            pltpu.VMEM((tl, S), jnp.float32),
        ],
        compiler_params=pltpu.CompilerParams(
            dimension_semantics=("parallel", "parallel")),
    )(query, W_q, kp, bias, mem2d)
    return ret, attn
